# baseline (speedup 1.0000x reference)
import functools

import jax
import jax.numpy as jnp
from jax import lax
from jax.experimental import pallas as pl
from jax.experimental.pallas import tpu as pltpu
from jax.experimental.pallas import tpu_sc as plsc

N_CAT = 26
VOCAB = 100000
D_MODEL = 32
BATCH = 16384

_NC = 2
_NS = 16

_QB = BATCH // 4
_LANES = 16
_SPLIT = 49920
_MID = 49920
_TAIL = VOCAB - _SPLIT - _MID
_BLEN = _MID + _TAIL


@functools.partial(
    pl.kernel,
    out_type=jax.ShapeDtypeStruct((N_CAT, D_MODEL, BATCH), jnp.float32),
    mesh=plsc.VectorSubcoreMesh(core_axis_name="c", subcore_axis_name="s"),
    compiler_params=pltpu.CompilerParams(needs_layout_passes=False),
    scratch_types=[
        pltpu.VMEM((_SPLIT,), jnp.float32),
        pltpu.VMEM((_BLEN,), jnp.float32),
        pltpu.VMEM((_TAIL,), jnp.float32),
        pltpu.VMEM((2, _QB), jnp.int32),
        pltpu.VMEM((2, _QB), jnp.float32),
        pltpu.SemaphoreType.DMA,
        pltpu.SemaphoreType.DMA,
        pltpu.SemaphoreType.DMA,
        pltpu.SemaphoreType.DMA,
    ],
)
def _gather_kernel(x_hbm, tab_hbm, tail_hbm, out_hbm,
                   rowa_v, rowb_v, rowt_v, idx_v, out_v, asem, bsem, isem,
                   osem):
    d = lax.axis_index("s") * _NC + lax.axis_index("c")
    _UNROLL = 8

    def _start_a(c):
        return pltpu.async_copy(
            tab_hbm.at[c, d, pl.ds(0, _SPLIT)], rowa_v, asem)

    def _start_b(c):
        pltpu.async_copy(tab_hbm.at[c, d, pl.ds(_SPLIT, _MID)],
                         rowb_v.at[pl.ds(0, _MID)], bsem)
        pltpu.async_copy(tail_hbm.at[c, d, :], rowt_v, bsem)

    def _wait_a(c):
        pltpu.make_async_copy(
            tab_hbm.at[c, d, pl.ds(0, _SPLIT)], rowa_v, asem).wait()

    def _wait_b(c):
        pltpu.make_async_copy(tab_hbm.at[c, d, pl.ds(_SPLIT, _MID)],
                              rowb_v.at[pl.ds(0, _MID)], bsem).wait()
        pltpu.make_async_copy(tail_hbm.at[c, d, :], rowt_v, bsem).wait()
        for k in range(_TAIL // _LANES):
            s = pl.ds(k * _LANES, _LANES)
            rowb_v[pl.ds(_MID + k * _LANES, _LANES)] = rowt_v[s]

    def _pass_a(b):
        def _g(j, _):
            for u in range(_UNROLL):
                s = pl.ds((j * _UNROLL + u) * _LANES, _LANES)
                iv = idx_v[b, s]
                out_v[b, s] = plsc.load_gather(
                    rowa_v, [jnp.minimum(iv, _SPLIT - 1)])
            return 0

        lax.fori_loop(0, _QB // (_LANES * _UNROLL), _g, 0)

    def _pass_b(b):
        def _g(j, _):
            for u in range(_UNROLL):
                s = pl.ds((j * _UNROLL + u) * _LANES, _LANES)
                iv = idx_v[b, s]
                g = plsc.load_gather(
                    rowb_v, [jnp.maximum(iv - _SPLIT, 0)])
                out_v[b, s] = jnp.where(iv >= _SPLIT, g, out_v[b, s])
            return 0

        lax.fori_loop(0, _QB // (_LANES * _UNROLL), _g, 0)

    def _per_cat(c, _):
        i_first = pltpu.async_copy(x_hbm.at[c, pl.ds(0, _QB)], idx_v.at[0],
                                   isem)
        i_first.wait()
        _wait_a(c)
        outs = {}
        for q in range(4):
            b = q % 2
            if q < 3:
                i_next = pltpu.async_copy(
                    x_hbm.at[c, pl.ds((q + 1) * _QB, _QB)],
                    idx_v.at[(q + 1) % 2], isem)
            if q >= 2:
                outs[q - 2].wait()
            _pass_a(b)
            if q == 0:
                _wait_b(c)
            if q == 3:
                def _pre_a():
                    _start_a(c + 1)

                pl.when(c < N_CAT - 1)(_pre_a)
            _pass_b(b)
            outs[q] = pltpu.async_copy(
                out_v.at[b], out_hbm.at[c, d, pl.ds(q * _QB, _QB)], osem)
            if q < 3:
                i_next.wait()
        def _pre_b():
            _start_b(c + 1)

        pl.when(c < N_CAT - 1)(_pre_b)
        outs[2].wait()
        outs[3].wait()
        return 0

    _start_a(0)
    _start_b(0)
    lax.fori_loop(0, N_CAT, _per_cat, 0)


def kernel(x, tables):
    x_t = x.T.astype(jnp.int32)
    tab_t = jnp.transpose(tables, (0, 2, 1))
    tail_t = jnp.transpose(tables[:, _SPLIT + _MID:, :], (0, 2, 1))
    out_t = _gather_kernel(x_t, tab_t, tail_t)
    return jnp.transpose(out_t, (2, 0, 1))


# speedup vs baseline: 1.3594x; 1.3594x over previous
import functools

import jax
import jax.numpy as jnp
from jax import lax
from jax.experimental import pallas as pl
from jax.experimental.pallas import tpu as pltpu
from jax.experimental.pallas import tpu_sc as plsc

N_CAT = 26
VOCAB = 100000
D_MODEL = 32
BATCH = 16384

_NC = 2
_NS = 16
_NW = _NC * _NS

_QB = BATCH // 4
_LANES = 16


@functools.partial(
    pl.kernel,
    out_type=jax.ShapeDtypeStruct((N_CAT, D_MODEL, BATCH), jnp.float32),
    mesh=plsc.VectorSubcoreMesh(core_axis_name="c", subcore_axis_name="s"),
    compiler_params=pltpu.CompilerParams(needs_layout_passes=False),
    scratch_types=[
        pltpu.VMEM((VOCAB,), jnp.float32),
        pltpu.VMEM((2, _QB), jnp.int32),
        pltpu.VMEM((2, _QB), jnp.float32),
        pltpu.SemaphoreType.DMA,
        pltpu.SemaphoreType.DMA,
        pltpu.SemaphoreType.DMA,
    ],
)
def _gather_kernel(x_hbm, tab_hbm, out_hbm, row_v, idx_v, out_v, rsem, isem, osem):
    d = lax.axis_index("s") * _NC + lax.axis_index("c")
    _UNROLL = 32

    def _gather_chunk(b):
        def _g(j, _):
            for u in range(_UNROLL):
                s = pl.ds((j * _UNROLL + u) * _LANES, _LANES)
                out_v[b, s] = plsc.load_gather(row_v, [idx_v[b, s]])
            return 0

        lax.fori_loop(0, _QB // (_LANES * _UNROLL), _g, 0)

    def _per_cat(c, _):
        r0 = pltpu.async_copy(tab_hbm.at[c, d, :], row_v, rsem)
        i_prev = pltpu.async_copy(x_hbm.at[c, pl.ds(0, _QB)], idx_v.at[0], isem)
        i_prev.wait()
        r0.wait()
        outs = {}
        for q in range(4):
            b = q % 2
            if q < 3:
                i_next = pltpu.async_copy(
                    x_hbm.at[c, pl.ds((q + 1) * _QB, _QB)],
                    idx_v.at[(q + 1) % 2], isem)
            if q >= 2:
                outs[q - 2].wait()
            _gather_chunk(b)
            outs[q] = pltpu.async_copy(
                out_v.at[b], out_hbm.at[c, d, pl.ds(q * _QB, _QB)], osem)
            if q < 3:
                i_next.wait()
        outs[2].wait()
        outs[3].wait()
        return 0

    lax.fori_loop(0, N_CAT, _per_cat, 0)


def kernel(x, tables):
    x_t = x.T.astype(jnp.int32)
    tab_t = jnp.transpose(tables, (0, 2, 1))
    out_t = _gather_kernel(x_t, tab_t)
    return jnp.transpose(out_t, (2, 0, 1))
